# baseline (speedup 1.0000x reference)
import functools

import jax
import jax.numpy as jnp
from jax import lax
from jax.experimental import pallas as pl
from jax.experimental.pallas import tpu as pltpu
from jax.experimental.pallas import tpu_sc as plsc

_N_GENES = 100000
_N_DH = 64
_N_LATENT = 50
_N_OI = 4096

_SC_CORES = 2
_SC_SUBCORES = 16
_NW = _SC_CORES * _SC_SUBCORES
_RPW = _N_OI // _NW


def _sc_gather_body(t_s, t_c, t_t, idx_hbm, out_s, out_c, out_t,
                    idx_v, idx_sm, comp, sem):
    wid = lax.axis_index("s") * _SC_CORES + lax.axis_index("c")
    base = wid * _RPW
    pltpu.sync_copy(idx_hbm.at[pl.ds(base, _RPW)], idx_v)
    pltpu.sync_copy(idx_v, idx_sm)
    for src, dst in ((t_s, out_s), (t_c, out_c), (t_t, out_t)):
        out_slice = dst.at[pl.ds(base, _RPW)]

        def fire(i, carry, src=src):
            g = idx_sm[i]
            t = lax.shift_right_logical(g, 3)
            r = lax.bitwise_and(g, 7)
            pltpu.make_async_copy(
                src.at[t, pl.ds(r, 1)], comp.at[pl.ds(i, 1)], sem).start()
            return carry

        lax.fori_loop(0, _RPW, fire, 0)
        pltpu.make_async_copy(out_slice, comp, sem).wait()
        pltpu.sync_copy(comp, out_slice)


@functools.cache
def _sc_gather():
  return pl.kernel(
    _sc_gather_body,
    out_type=[jax.ShapeDtypeStruct((_N_OI, _N_DH), jnp.float32)] * 3,
    mesh=plsc.VectorSubcoreMesh(
        core_axis_name="c", subcore_axis_name="s",
        num_cores=_SC_CORES, num_subcores=_SC_SUBCORES),
    scratch_types=[
        pltpu.VMEM((_RPW,), jnp.int32),
        pltpu.SMEM((_RPW,), jnp.int32),
        pltpu.VMEM((_RPW, _N_DH), jnp.float32),
        pltpu.SemaphoreType.DMA,
    ],
  )


def _height_body(s_ref, c_ref, t_ref, lat_ref, o_ref):
    e = jnp.exp(c_ref[...])
    x = e * lat_ref[...] + t_ref[...]
    o_ref[...] = s_ref[...] / (1.0 + jnp.exp(-x))


_HB = 4096


def _height(sf, cf, tf, lat2):
    n = _N_OI * _N_DH
    return pl.pallas_call(
        _height_body,
        grid=(n // _HB,),
        in_specs=[
            pl.BlockSpec((_HB, 1), lambda i: (i, 0)),
            pl.BlockSpec((_HB, 1), lambda i: (i, 0)),
            pl.BlockSpec((_HB, 1), lambda i: (i, 0)),
            pl.BlockSpec((1, _N_LATENT), lambda i: (0, 0)),
        ],
        out_specs=pl.BlockSpec((_HB, _N_LATENT), lambda i: (i, 0)),
        out_shape=jax.ShapeDtypeStruct((n, _N_LATENT), jnp.float32),
    )(sf, cf, tf, lat2)


def _overall_body(w_ref, lat_ref, o_ref):
    o_ref[...] = w_ref[...] * lat_ref[...]


_OB = 10000


def _overall(w, lat2):
    return pl.pallas_call(
        _overall_body,
        grid=(_N_GENES // _OB,),
        in_specs=[
            pl.BlockSpec((_OB, 1), lambda i: (i, 0)),
            pl.BlockSpec((1, _N_LATENT), lambda i: (0, 0)),
        ],
        out_specs=pl.BlockSpec((_OB, _N_LATENT), lambda i: (i, 0)),
        out_shape=jax.ShapeDtypeStruct((_N_GENES, _N_LATENT), jnp.float32),
    )(w, lat2)


def kernel(latent, genes_oi, W_height_slope, W_height_scale, W_height_shift,
           W_overall_slope):
    g_s = jnp.take(W_height_slope, genes_oi, axis=0)
    g_c = jnp.take(W_height_scale, genes_oi, axis=0)
    g_t = jnp.take(W_height_shift, genes_oi, axis=0)
    n = _N_OI * _N_DH
    lat2 = latent.reshape(1, _N_LATENT)
    dh = _height(g_s.reshape(n, 1), g_c.reshape(n, 1), g_t.reshape(n, 1),
                 lat2)
    do = _overall(W_overall_slope, lat2)
    return (dh.reshape(_N_OI, _N_DH, _N_LATENT),
            do.reshape(_N_GENES, 1, _N_LATENT))


# speedup vs baseline: 1.6478x; 1.6478x over previous
import functools

import jax
import jax.numpy as jnp
from jax import lax
from jax.experimental import pallas as pl
from jax.experimental.pallas import tpu as pltpu
from jax.experimental.pallas import tpu_sc as plsc

_N_GENES = 100000
_N_DH = 64
_N_LATENT = 50
_N_OI = 4096

_SC_CORES = 2
_SC_SUBCORES = 16
_NW = _SC_CORES * _SC_SUBCORES
_RPW = _N_OI // _NW
_CHUNK = 64
_L = 16


_GPS = _N_OI // _SC_CORES
_ICH = 256


def _sc_gather_body(t_s, t_c, t_t, idx_hbm, out_s, out_c, out_t,
                    idx_sm, sem):
    cid = lax.axis_index("c")
    base = cid * _GPS
    for ch in range(_GPS // _ICH):
        pltpu.sync_copy(idx_hbm.at[pl.ds(base + ch * _ICH, _ICH)], idx_sm)
        for src, dst in ((t_s, out_s), (t_c, out_c), (t_t, out_t)):

            def fire(i, carry, src=src, dst=dst, ch=ch):
                g = idx_sm[i]
                t = lax.shift_right_logical(g, 3)
                r = lax.bitwise_and(g, 7)
                pltpu.make_async_copy(
                    src.at[t, pl.ds(r, 1)],
                    dst.at[pl.ds(base + ch * _ICH + i, 1)], sem).start()
                return carry

            def drain(i, carry, src=src, dst=dst, ch=ch):
                g = idx_sm[i]
                t = lax.shift_right_logical(g, 3)
                r = lax.bitwise_and(g, 7)
                pltpu.make_async_copy(
                    src.at[t, pl.ds(r, 1)],
                    dst.at[pl.ds(base + ch * _ICH + i, 1)], sem).wait()
                return carry

            lax.fori_loop(0, _ICH, fire, 0)
            lax.fori_loop(0, _ICH, drain, 0)


@functools.cache
def _sc_gather():
  return pl.kernel(
    _sc_gather_body,
    out_type=[jax.ShapeDtypeStruct((_N_OI, _N_DH), jnp.float32)] * 3,
    mesh=plsc.ScalarSubcoreMesh(axis_name="c", num_cores=_SC_CORES),
    scratch_types=[
        pltpu.SMEM((_ICH,), jnp.int32),
        pltpu.SemaphoreType.DMA,
    ],
  )


def _height_body(s_ref, c_ref, t_ref, lat_ref, o_ref):
    e = jnp.exp(c_ref[...])[:, :, None]
    t = t_ref[...][:, :, None]
    s = s_ref[...][:, :, None]
    lat = lat_ref[...][None, :, :]
    x = e * lat + t
    o_ref[...] = s / (1.0 + jnp.exp(-x))


_HB = 256


def _height(gs, gc, gt, lat2):
    return pl.pallas_call(
        _height_body,
        grid=(_N_OI // _HB,),
        in_specs=[
            pl.BlockSpec((_HB, _N_DH), lambda i: (i, 0)),
            pl.BlockSpec((_HB, _N_DH), lambda i: (i, 0)),
            pl.BlockSpec((_HB, _N_DH), lambda i: (i, 0)),
            pl.BlockSpec((1, _N_LATENT), lambda i: (0, 0)),
        ],
        out_specs=pl.BlockSpec((_HB, _N_DH, _N_LATENT), lambda i: (i, 0, 0)),
        out_shape=jax.ShapeDtypeStruct((_N_OI, _N_DH, _N_LATENT),
                                       jnp.float32),
    )(gs, gc, gt, lat2)


def _overall_body(w_ref, lat_ref, o_ref):
    o_ref[...] = w_ref[...] * lat_ref[...]


_OB = 10000


def _overall(w, lat2):
    return pl.pallas_call(
        _overall_body,
        grid=(_N_GENES // _OB,),
        in_specs=[
            pl.BlockSpec((_OB, 1), lambda i: (i, 0)),
            pl.BlockSpec((1, _N_LATENT), lambda i: (0, 0)),
        ],
        out_specs=pl.BlockSpec((_OB, _N_LATENT), lambda i: (i, 0)),
        out_shape=jax.ShapeDtypeStruct((_N_GENES, _N_LATENT), jnp.float32),
    )(w, lat2)


def kernel(latent, genes_oi, W_height_slope, W_height_scale, W_height_shift,
           W_overall_slope):
    v3 = lambda w: w.reshape(_N_GENES // 8, 8, _N_DH)
    g_s, g_c, g_t = _sc_gather()(
        v3(W_height_slope), v3(W_height_scale), v3(W_height_shift), genes_oi)
    lat2 = latent.reshape(1, _N_LATENT)
    dh = _height(g_s, g_c, g_t, lat2)
    do = _overall(W_overall_slope, lat2)
    return (dh, do.reshape(_N_GENES, 1, _N_LATENT))


# speedup vs baseline: 2.5702x; 1.5598x over previous
import functools

import jax
import jax.numpy as jnp
from jax import lax
from jax.experimental import pallas as pl
from jax.experimental.pallas import tpu as pltpu
from jax.experimental.pallas import tpu_sc as plsc

_N_GENES = 100000
_N_DH = 64
_N_LATENT = 50
_N_OI = 4096

_SC_CORES = 2
_GPS = _N_OI // _SC_CORES
_ICH = 512


def _sc_gather_body(t_s, t_c, t_t, idx_hbm, out_s, out_c, out_t,
                    idx_sm, sem):
    cid = lax.axis_index("c")
    base = cid * _GPS
    for ch in range(_GPS // _ICH):
        pltpu.sync_copy(idx_hbm.at[pl.ds(base + ch * _ICH, _ICH)], idx_sm)
        for src, dst in ((t_s, out_s), (t_c, out_c), (t_t, out_t)):

            def fire(i, carry, src=src, dst=dst, ch=ch):
                g = idx_sm[i]
                t = lax.shift_right_logical(g, 3)
                r = lax.bitwise_and(g, 7)
                pltpu.make_async_copy(
                    src.at[t, pl.ds(r, 1)],
                    dst.at[pl.ds(base + ch * _ICH + i, 1)], sem).start()
                return carry

            lax.fori_loop(0, _ICH, fire, 0)
    for dst in (out_s, out_c, out_t):
        pltpu.make_async_copy(
            dst.at[pl.ds(0, _GPS)], dst.at[pl.ds(base, _GPS)], sem).wait()


@functools.cache
def _sc_gather():
  return pl.kernel(
    _sc_gather_body,
    out_type=[jax.ShapeDtypeStruct((_N_OI, _N_DH), jnp.float32)] * 3,
    mesh=plsc.ScalarSubcoreMesh(axis_name="c", num_cores=_SC_CORES),
    scratch_types=[
        pltpu.SMEM((_ICH,), jnp.int32),
        pltpu.SemaphoreType.DMA,
    ],
  )


def _height_body(s_ref, c_ref, t_ref, lat_ref, o_ref):
    e = jnp.exp(c_ref[...])[None, :, :]
    t = t_ref[...][None, :, :]
    s = s_ref[...][None, :, :]
    lat = lat_ref[...][:, :, None]
    x = e * lat + t
    o_ref[...] = s / (1.0 + jnp.exp(-x))


_HB = 512


def _height(gst, gct, gtt, lat2):
    return pl.pallas_call(
        _height_body,
        grid=(_N_OI // _HB,),
        in_specs=[
            pl.BlockSpec((_N_DH, _HB), lambda i: (0, i)),
            pl.BlockSpec((_N_DH, _HB), lambda i: (0, i)),
            pl.BlockSpec((_N_DH, _HB), lambda i: (0, i)),
            pl.BlockSpec((_N_LATENT, 1), lambda i: (0, 0)),
        ],
        out_specs=pl.BlockSpec((_N_LATENT, _N_DH, _HB),
                               lambda i: (0, 0, i)),
        out_shape=jax.ShapeDtypeStruct((_N_LATENT, _N_DH, _N_OI),
                                       jnp.float32),
    )(gst, gct, gtt, lat2)


def _overall_body(w_ref, lat_ref, o_ref):
    o_ref[...] = (w_ref[...] * lat_ref[..., 0])[None]


def _overall(w2, lat3):
    return pl.pallas_call(
        _overall_body,
        grid=(_N_LATENT,),
        in_specs=[
            pl.BlockSpec((1, _N_GENES), lambda i: (0, 0)),
            pl.BlockSpec((1, 1, 1), lambda i: (i, 0, 0)),
        ],
        out_specs=pl.BlockSpec((1, 1, _N_GENES), lambda i: (i, 0, 0)),
        out_shape=jax.ShapeDtypeStruct((_N_LATENT, 1, _N_GENES),
                                       jnp.float32),
    )(w2, lat3)


def kernel(latent, genes_oi, W_height_slope, W_height_scale, W_height_shift,
           W_overall_slope):
    v3 = lambda w: w.reshape(_N_GENES // 8, 8, _N_DH)
    g_s, g_c, g_t = _sc_gather()(
        v3(W_height_slope), v3(W_height_scale), v3(W_height_shift), genes_oi)
    lat2 = latent.reshape(_N_LATENT, 1)
    dh_t = _height(g_s.T, g_c.T, g_t.T, lat2)
    do_f = _overall(W_overall_slope.reshape(1, _N_GENES),
                    latent.reshape(_N_LATENT, 1, 1))
    dh = jnp.transpose(dh_t, (2, 1, 0))
    do = jnp.transpose(do_f, (2, 1, 0))
    return (dh, do)


# speedup vs baseline: 2.5704x; 1.0001x over previous
import functools

import jax
import jax.numpy as jnp
from jax import lax
from jax.experimental import pallas as pl
from jax.experimental.pallas import tpu as pltpu
from jax.experimental.pallas import tpu_sc as plsc

_N_GENES = 100000
_N_DH = 64
_N_LATENT = 50
_N_OI = 4096

_SC_CORES = 2
_GPS = _N_OI // _SC_CORES
_ICH = 512


def _sc_gather_body(t_s, t_c, t_t, idx_hbm, out_s, out_c, out_t,
                    idx_sm, sem_s, sem_c, sem_t):
    cid = lax.axis_index("c")
    base = cid * _GPS
    sems = (sem_s, sem_c, sem_t)
    for ch in range(_GPS // _ICH):
        pltpu.sync_copy(idx_hbm.at[pl.ds(base + ch * _ICH, _ICH)], idx_sm)
        for (src, dst), sem in zip(((t_s, out_s), (t_c, out_c), (t_t, out_t)),
                                   sems):

            def fire(i, carry, src=src, dst=dst, ch=ch, sem=sem):
                g = idx_sm[i]
                t = lax.shift_right_logical(g, 3)
                r = lax.bitwise_and(g, 7)
                pltpu.make_async_copy(
                    src.at[t, pl.ds(r, 1)],
                    dst.at[pl.ds(base + ch * _ICH + i, 1)], sem).start()
                return carry

            lax.fori_loop(0, _ICH, fire, 0)
    for dst, sem in zip((out_s, out_c, out_t), sems):
        pltpu.make_async_copy(
            dst.at[pl.ds(0, _GPS)], dst.at[pl.ds(base, _GPS)], sem).wait()


@functools.cache
def _sc_gather():
  return pl.kernel(
    _sc_gather_body,
    out_type=[jax.ShapeDtypeStruct((_N_OI, _N_DH), jnp.float32)] * 3,
    mesh=plsc.ScalarSubcoreMesh(axis_name="c", num_cores=_SC_CORES),
    scratch_types=[
        pltpu.SMEM((_ICH,), jnp.int32),
        pltpu.SemaphoreType.DMA,
        pltpu.SemaphoreType.DMA,
        pltpu.SemaphoreType.DMA,
    ],
  )


def _height_body(s_ref, c_ref, t_ref, lat_ref, o_ref):
    e = jnp.exp(c_ref[...])[None, :, :]
    t = t_ref[...][None, :, :]
    s = s_ref[...][None, :, :]
    lat = lat_ref[...][:, :, None]
    x = e * lat + t
    o_ref[...] = s / (1.0 + jnp.exp(-x))


_HB = 512


def _height(gst, gct, gtt, lat2):
    return pl.pallas_call(
        _height_body,
        grid=(_N_OI // _HB,),
        in_specs=[
            pl.BlockSpec((_N_DH, _HB), lambda i: (0, i)),
            pl.BlockSpec((_N_DH, _HB), lambda i: (0, i)),
            pl.BlockSpec((_N_DH, _HB), lambda i: (0, i)),
            pl.BlockSpec((_N_LATENT, 1), lambda i: (0, 0)),
        ],
        out_specs=pl.BlockSpec((_N_LATENT, _N_DH, _HB),
                               lambda i: (0, 0, i)),
        out_shape=jax.ShapeDtypeStruct((_N_LATENT, _N_DH, _N_OI),
                                       jnp.float32),
    )(gst, gct, gtt, lat2)


def _overall_body(w_ref, lat_ref, o_ref):
    o_ref[...] = w_ref[...][None] * lat_ref[...]


_OBL = 10


def _overall(w2, lat3):
    return pl.pallas_call(
        _overall_body,
        grid=(_N_LATENT // _OBL,),
        in_specs=[
            pl.BlockSpec((1, _N_GENES), lambda i: (0, 0)),
            pl.BlockSpec((_OBL, 1, 1), lambda i: (i, 0, 0)),
        ],
        out_specs=pl.BlockSpec((_OBL, 1, _N_GENES), lambda i: (i, 0, 0)),
        out_shape=jax.ShapeDtypeStruct((_N_LATENT, 1, _N_GENES),
                                       jnp.float32),
    )(w2, lat3)


def kernel(latent, genes_oi, W_height_slope, W_height_scale, W_height_shift,
           W_overall_slope):
    v3 = lambda w: w.reshape(_N_GENES // 8, 8, _N_DH)
    g_s, g_c, g_t = _sc_gather()(
        v3(W_height_slope), v3(W_height_scale), v3(W_height_shift), genes_oi)
    lat2 = latent.reshape(_N_LATENT, 1)
    dh_t = _height(g_s.T, g_c.T, g_t.T, lat2)
    do_f = _overall(W_overall_slope.reshape(1, _N_GENES),
                    latent.reshape(_N_LATENT, 1, 1))
    dh = jnp.transpose(dh_t, (2, 1, 0))
    do = jnp.transpose(do_f, (2, 1, 0))
    return (dh, do)


# speedup vs baseline: 3.5338x; 1.3748x over previous
import functools

import jax
import jax.numpy as jnp
from jax import lax
from jax.experimental import pallas as pl
from jax.experimental.pallas import tpu as pltpu
from jax.experimental.pallas import tpu_sc as plsc

_N_GENES = 100000
_N_DH = 64
_N_LATENT = 50
_N_OI = 4096

_SC_CORES = 2
_SC_SUBCORES = 16
_NW = _SC_CORES * _SC_SUBCORES
_RPW = _N_OI // _NW


def _sc_gather_body(t_s, t_c, t_t, idx_hbm, out_s, out_c, out_t,
                    idx_v, rows_s, rows_c, rows_t, sem_s, sem_c, sem_t):
    wid = lax.axis_index("s") * _SC_CORES + lax.axis_index("c")
    base = wid * _RPW
    pltpu.sync_copy(idx_hbm.at[pl.ds(base, _RPW)], idx_v)
    c1 = pltpu.async_copy(t_s.at[idx_v], rows_s, sem_s)
    c2 = pltpu.async_copy(t_c.at[idx_v], rows_c, sem_c)
    c3 = pltpu.async_copy(t_t.at[idx_v], rows_t, sem_t)
    c1.wait()
    c2.wait()
    c3.wait()
    pltpu.sync_copy(rows_s, out_s.at[pl.ds(base, _RPW)])
    pltpu.sync_copy(rows_c, out_c.at[pl.ds(base, _RPW)])
    pltpu.sync_copy(rows_t, out_t.at[pl.ds(base, _RPW)])


@functools.cache
def _sc_gather():
  return pl.kernel(
    _sc_gather_body,
    out_type=[jax.ShapeDtypeStruct((_N_OI, _N_DH), jnp.float32)] * 3,
    mesh=plsc.VectorSubcoreMesh(
        core_axis_name="c", subcore_axis_name="s",
        num_cores=_SC_CORES, num_subcores=_SC_SUBCORES),
    scratch_types=[
        pltpu.VMEM((_RPW,), jnp.int32),
        pltpu.VMEM((_RPW, _N_DH), jnp.float32),
        pltpu.VMEM((_RPW, _N_DH), jnp.float32),
        pltpu.VMEM((_RPW, _N_DH), jnp.float32),
        pltpu.SemaphoreType.DMA,
        pltpu.SemaphoreType.DMA,
        pltpu.SemaphoreType.DMA,
    ],
    compiler_params=pltpu.CompilerParams(use_tc_tiling_on_sc=False),
  )


def _height_body(s_ref, c_ref, t_ref, lat_ref, o_ref):
    e = jnp.exp(c_ref[...])[None, :, :]
    t = t_ref[...][None, :, :]
    s = s_ref[...][None, :, :]
    lat = lat_ref[...][:, :, None]
    x = e * lat + t
    o_ref[...] = s / (1.0 + jnp.exp(-x))


_HB = 512


def _height(gst, gct, gtt, lat2):
    return pl.pallas_call(
        _height_body,
        grid=(_N_OI // _HB,),
        in_specs=[
            pl.BlockSpec((_N_DH, _HB), lambda i: (0, i)),
            pl.BlockSpec((_N_DH, _HB), lambda i: (0, i)),
            pl.BlockSpec((_N_DH, _HB), lambda i: (0, i)),
            pl.BlockSpec((_N_LATENT, 1), lambda i: (0, 0)),
        ],
        out_specs=pl.BlockSpec((_N_LATENT, _N_DH, _HB),
                               lambda i: (0, 0, i)),
        out_shape=jax.ShapeDtypeStruct((_N_LATENT, _N_DH, _N_OI),
                                       jnp.float32),
    )(gst, gct, gtt, lat2)


def _overall_body(w_ref, lat_ref, o_ref):
    o_ref[...] = w_ref[...][None] * lat_ref[...]


_OBL = 10


def _overall(w2, lat3):
    return pl.pallas_call(
        _overall_body,
        grid=(_N_LATENT // _OBL,),
        in_specs=[
            pl.BlockSpec((1, _N_GENES), lambda i: (0, 0)),
            pl.BlockSpec((_OBL, 1, 1), lambda i: (i, 0, 0)),
        ],
        out_specs=pl.BlockSpec((_OBL, 1, _N_GENES), lambda i: (i, 0, 0)),
        out_shape=jax.ShapeDtypeStruct((_N_LATENT, 1, _N_GENES),
                                       jnp.float32),
    )(w2, lat3)


def kernel(latent, genes_oi, W_height_slope, W_height_scale, W_height_shift,
           W_overall_slope):
    g_s, g_c, g_t = _sc_gather()(
        W_height_slope, W_height_scale, W_height_shift, genes_oi)
    lat2 = latent.reshape(_N_LATENT, 1)
    dh_t = _height(g_s.T, g_c.T, g_t.T, lat2)
    do_f = _overall(W_overall_slope.reshape(1, _N_GENES),
                    latent.reshape(_N_LATENT, 1, 1))
    dh = jnp.transpose(dh_t, (2, 1, 0))
    do = jnp.transpose(do_f, (2, 1, 0))
    return (dh, do)


# speedup vs baseline: 4.1832x; 1.1838x over previous
import functools

import jax
import jax.numpy as jnp
from jax import lax
from jax.experimental import pallas as pl
from jax.experimental.pallas import tpu as pltpu
from jax.experimental.pallas import tpu_sc as plsc

_N_GENES = 100000
_N_DH = 64
_N_LATENT = 50
_N_OI = 4096

_SC_CORES = 2
_SC_SUBCORES = 16
_NW = _SC_CORES * _SC_SUBCORES
_RPW = _N_OI // _NW


_L = 16


def _sc_gather_body(t_s, t_c, t_t, idx_hbm, out_s, out_c, out_t,
                    gidx, idxf, rows_s, rows_c, rows_t,
                    sem_s, sem_c, sem_t):
    wid = lax.axis_index("s") * _SC_CORES + lax.axis_index("c")
    base = wid * _RPW
    pltpu.sync_copy(idx_hbm.at[pl.ds(base, _RPW)], gidx)

    def build(h, carry):
        for k in range(_RPW // _L):
            idxf[pl.ds(h * _RPW + k * _L, _L)] = (
                gidx[pl.ds(k * _L, _L)] + h * _N_GENES)
        return carry

    lax.fori_loop(0, _N_DH, build, 0)
    c1 = pltpu.async_copy(t_s.at[idxf], rows_s, sem_s)
    c2 = pltpu.async_copy(t_c.at[idxf], rows_c, sem_c)
    c3 = pltpu.async_copy(t_t.at[idxf], rows_t, sem_t)
    c1.wait()
    c2.wait()
    c3.wait()
    for rows, dst, sem in ((rows_s, out_s, sem_s), (rows_c, out_c, sem_c),
                           (rows_t, out_t, sem_t)):

        def put(h, carry, rows=rows, dst=dst, sem=sem):
            pltpu.make_async_copy(
                rows.at[pl.ds(h * _RPW, _RPW)],
                dst.at[pl.ds(h * _N_OI + base, _RPW)], sem).start()
            return carry

        lax.fori_loop(0, _N_DH, put, 0)
    for rows, dst, sem in ((rows_s, out_s, sem_s), (rows_c, out_c, sem_c),
                           (rows_t, out_t, sem_t)):
        pltpu.make_async_copy(dst.at[pl.ds(0, _N_DH * _RPW)], rows,
                              sem).wait()


@functools.cache
def _sc_gather():
  n = _N_DH * _N_OI
  return pl.kernel(
    _sc_gather_body,
    out_type=[jax.ShapeDtypeStruct((n,), jnp.float32)] * 3,
    mesh=plsc.VectorSubcoreMesh(
        core_axis_name="c", subcore_axis_name="s",
        num_cores=_SC_CORES, num_subcores=_SC_SUBCORES),
    scratch_types=[
        pltpu.VMEM((_RPW,), jnp.int32),
        pltpu.VMEM((_N_DH * _RPW,), jnp.int32),
        pltpu.VMEM((_N_DH * _RPW,), jnp.float32),
        pltpu.VMEM((_N_DH * _RPW,), jnp.float32),
        pltpu.VMEM((_N_DH * _RPW,), jnp.float32),
        pltpu.SemaphoreType.DMA,
        pltpu.SemaphoreType.DMA,
        pltpu.SemaphoreType.DMA,
    ],
    compiler_params=pltpu.CompilerParams(use_tc_tiling_on_sc=False),
  )


def _height_body(s_ref, c_ref, t_ref, lat_ref, o_ref):
    e = jnp.exp(c_ref[...])[None, :, :]
    t = t_ref[...][None, :, :]
    s = s_ref[...][None, :, :]
    lat = lat_ref[...][:, :, None]
    x = e * lat + t
    o_ref[...] = s / (1.0 + jnp.exp(-x))


_HB = 512


def _height(gst, gct, gtt, lat2):
    return pl.pallas_call(
        _height_body,
        grid=(_N_OI // _HB,),
        in_specs=[
            pl.BlockSpec((_N_DH, _HB), lambda i: (0, i)),
            pl.BlockSpec((_N_DH, _HB), lambda i: (0, i)),
            pl.BlockSpec((_N_DH, _HB), lambda i: (0, i)),
            pl.BlockSpec((_N_LATENT, 1), lambda i: (0, 0)),
        ],
        out_specs=pl.BlockSpec((_N_LATENT, _N_DH, _HB),
                               lambda i: (0, 0, i)),
        out_shape=jax.ShapeDtypeStruct((_N_LATENT, _N_DH, _N_OI),
                                       jnp.float32),
    )(gst, gct, gtt, lat2)


def _overall_body(w_ref, lat_ref, o_ref):
    o_ref[...] = w_ref[...][None] * lat_ref[...]


_OBL = 10


def _overall(w2, lat3):
    return pl.pallas_call(
        _overall_body,
        grid=(_N_LATENT // _OBL,),
        in_specs=[
            pl.BlockSpec((1, _N_GENES), lambda i: (0, 0)),
            pl.BlockSpec((_OBL, 1, 1), lambda i: (i, 0, 0)),
        ],
        out_specs=pl.BlockSpec((_OBL, 1, _N_GENES), lambda i: (i, 0, 0)),
        out_shape=jax.ShapeDtypeStruct((_N_LATENT, 1, _N_GENES),
                                       jnp.float32),
    )(w2, lat3)


def kernel(latent, genes_oi, W_height_slope, W_height_scale, W_height_shift,
           W_overall_slope):
    hflat = lambda w: w.T.reshape(_N_GENES * _N_DH)
    g_s, g_c, g_t = _sc_gather()(
        hflat(W_height_slope), hflat(W_height_scale), hflat(W_height_shift),
        genes_oi)
    lat2 = latent.reshape(_N_LATENT, 1)
    v2 = lambda g: g.reshape(_N_DH, _N_OI)
    dh_t = _height(v2(g_s), v2(g_c), v2(g_t), lat2)
    do_f = _overall(W_overall_slope.reshape(1, _N_GENES),
                    latent.reshape(_N_LATENT, 1, 1))
    dh = jnp.transpose(dh_t, (2, 1, 0))
    do = jnp.transpose(do_f, (2, 1, 0))
    return (dh, do)


# speedup vs baseline: 4.7275x; 1.1301x over previous
import functools

import jax
import jax.numpy as jnp
from jax import lax
from jax.experimental import pallas as pl
from jax.experimental.pallas import tpu as pltpu
from jax.experimental.pallas import tpu_sc as plsc

_N_GENES = 100000
_N_DH = 64
_N_LATENT = 50
_N_OI = 4096

_SC_CORES = 2
_SC_SUBCORES = 16
_NW = _SC_CORES * _SC_SUBCORES
_RPW = _N_OI // _NW


_L = 16


def _sc_gather_body(tbl, idx_hbm, out, gidx, idxf, rows, sem):
    wid = lax.axis_index("s") * _SC_CORES + lax.axis_index("c")
    base = wid * _RPW
    pltpu.sync_copy(idx_hbm.at[pl.ds(base, _RPW)], gidx)

    def build(h, carry):
        for k in range(_RPW // _L):
            idxf[pl.ds(h * _RPW + k * _L, _L)] = (
                gidx[pl.ds(k * _L, _L)] + h * _N_GENES)
        return carry

    lax.fori_loop(0, _N_DH, build, 0)
    pltpu.async_copy(tbl.at[idxf], rows, sem).wait()

    def put(h, carry):
        pltpu.make_async_copy(
            rows.at[pl.ds(h * _RPW, _RPW)],
            out.at[pl.ds(h * _N_OI + base, _RPW)], sem).start()
        return carry

    lax.fori_loop(0, _N_DH, put, 0)
    pltpu.make_async_copy(out.at[pl.ds(0, _N_DH * _RPW)], rows, sem).wait()


@functools.cache
def _sc_gather():
  n = _N_DH * _N_OI
  return pl.kernel(
    _sc_gather_body,
    out_type=jax.ShapeDtypeStruct((n,), jnp.float32),
    mesh=plsc.VectorSubcoreMesh(
        core_axis_name="c", subcore_axis_name="s",
        num_cores=_SC_CORES, num_subcores=_SC_SUBCORES),
    scratch_types=[
        pltpu.VMEM((_RPW,), jnp.int32),
        pltpu.VMEM((_N_DH * _RPW,), jnp.int32),
        pltpu.VMEM((_N_DH * _RPW,), jnp.float32),
        pltpu.SemaphoreType.DMA,
    ],
    compiler_params=pltpu.CompilerParams(use_tc_tiling_on_sc=False),
  )


def _height_body(s_ref, c_ref, t_ref, lat_ref, o_ref):
    e = jnp.exp(c_ref[...])[None, :, :]
    t = t_ref[...][None, :, :]
    s = s_ref[...][None, :, :]
    lat = lat_ref[...][:, :, None]
    x = e * lat + t
    o_ref[...] = s / (1.0 + jnp.exp(-x))


_HB = 512


def _height(gst, gct, gtt, lat2):
    return pl.pallas_call(
        _height_body,
        grid=(_N_OI // _HB,),
        in_specs=[
            pl.BlockSpec((_N_DH, _HB), lambda i: (0, i)),
            pl.BlockSpec((_N_DH, _HB), lambda i: (0, i)),
            pl.BlockSpec((_N_DH, _HB), lambda i: (0, i)),
            pl.BlockSpec((_N_LATENT, 1), lambda i: (0, 0)),
        ],
        out_specs=pl.BlockSpec((_N_LATENT, _N_DH, _HB),
                               lambda i: (0, 0, i)),
        out_shape=jax.ShapeDtypeStruct((_N_LATENT, _N_DH, _N_OI),
                                       jnp.float32),
    )(gst, gct, gtt, lat2)


def _overall_body(w_ref, lat_ref, o_ref):
    o_ref[...] = w_ref[...][None] * lat_ref[...]


_OBL = 10


def _overall(w2, lat3):
    return pl.pallas_call(
        _overall_body,
        grid=(_N_LATENT // _OBL,),
        in_specs=[
            pl.BlockSpec((1, _N_GENES), lambda i: (0, 0)),
            pl.BlockSpec((_OBL, 1, 1), lambda i: (i, 0, 0)),
        ],
        out_specs=pl.BlockSpec((_OBL, 1, _N_GENES), lambda i: (i, 0, 0)),
        out_shape=jax.ShapeDtypeStruct((_N_LATENT, 1, _N_GENES),
                                       jnp.float32),
    )(w2, lat3)


def kernel(latent, genes_oi, W_height_slope, W_height_scale, W_height_shift,
           W_overall_slope):
    hflat = lambda w: w.T.reshape(_N_GENES * _N_DH)
    gather = _sc_gather()
    g_s = gather(hflat(W_height_slope), genes_oi)
    g_c = gather(hflat(W_height_scale), genes_oi)
    g_t = gather(hflat(W_height_shift), genes_oi)
    lat2 = latent.reshape(_N_LATENT, 1)
    v2 = lambda g: g.reshape(_N_DH, _N_OI)
    dh_t = _height(v2(g_s), v2(g_c), v2(g_t), lat2)
    do_f = _overall(W_overall_slope.reshape(1, _N_GENES),
                    latent.reshape(_N_LATENT, 1, 1))
    dh = jnp.transpose(dh_t, (2, 1, 0))
    do = jnp.transpose(do_f, (2, 1, 0))
    return (dh, do)


# speedup vs baseline: 5.7478x; 1.2158x over previous
import functools

import jax
import jax.numpy as jnp
from jax import lax
from jax.experimental import pallas as pl
from jax.experimental.pallas import tpu as pltpu
from jax.experimental.pallas import tpu_sc as plsc

_N_GENES = 100000
_N_DH = 64
_N_LATENT = 50
_N_OI = 4096

_SC_CORES = 2
_SC_SUBCORES = 16
_NW = _SC_CORES * _SC_SUBCORES
_RPW = _N_OI // _NW


_L = 16


def _sc_gather_body(tbl, idx_hbm, out, gidx, rows, sem):
    wid = lax.axis_index("s") * _SC_CORES + lax.axis_index("c")
    base = wid * _RPW
    pltpu.sync_copy(idx_hbm.at[pl.ds(base, _RPW)], gidx)
    lanes = lax.iota(jnp.int32, _L)

    def fire(i, carry):
        chunk = gidx[pl.ds((i // _L) * _L, _L)]
        g = jnp.sum(jnp.where(lanes == i % _L, chunk, 0))
        t = lax.shift_right_logical(g, 3)
        r = lax.bitwise_and(g, 7)
        pltpu.make_async_copy(
            tbl.at[t, pl.ds(r, 1)], rows.at[pl.ds(i, 1)], sem).start()
        return carry

    lax.fori_loop(0, _RPW, fire, 0)
    pltpu.make_async_copy(out.at[pl.ds(0, _RPW)], rows, sem).wait()
    pltpu.sync_copy(rows, out.at[pl.ds(base, _RPW)])


@functools.cache
def _sc_gather():
  return pl.kernel(
    _sc_gather_body,
    out_type=jax.ShapeDtypeStruct((_N_OI, _N_DH), jnp.float32),
    mesh=plsc.VectorSubcoreMesh(
        core_axis_name="c", subcore_axis_name="s",
        num_cores=_SC_CORES, num_subcores=_SC_SUBCORES),
    scratch_types=[
        pltpu.VMEM((_RPW,), jnp.int32),
        pltpu.VMEM((_RPW, _N_DH), jnp.float32),
        pltpu.SemaphoreType.DMA,
    ],
    compiler_params=pltpu.CompilerParams(needs_layout_passes=False),
  )


def _height_body(s_ref, c_ref, t_ref, lat_ref, o_ref):
    e = jnp.exp(c_ref[...])[None, :, :]
    t = t_ref[...][None, :, :]
    s = s_ref[...][None, :, :]
    lat = lat_ref[...][:, :, None]
    x = e * lat + t
    o_ref[...] = s / (1.0 + jnp.exp(-x))


_HB = 512


def _height(gst, gct, gtt, lat2):
    return pl.pallas_call(
        _height_body,
        grid=(_N_OI // _HB,),
        in_specs=[
            pl.BlockSpec((_N_DH, _HB), lambda i: (0, i)),
            pl.BlockSpec((_N_DH, _HB), lambda i: (0, i)),
            pl.BlockSpec((_N_DH, _HB), lambda i: (0, i)),
            pl.BlockSpec((_N_LATENT, 1), lambda i: (0, 0)),
        ],
        out_specs=pl.BlockSpec((_N_LATENT, _N_DH, _HB),
                               lambda i: (0, 0, i)),
        out_shape=jax.ShapeDtypeStruct((_N_LATENT, _N_DH, _N_OI),
                                       jnp.float32),
    )(gst, gct, gtt, lat2)


def _overall_body(w_ref, lat_ref, o_ref):
    o_ref[...] = w_ref[...][None] * lat_ref[...]


_OBL = 10


def _overall(w2, lat3):
    return pl.pallas_call(
        _overall_body,
        grid=(_N_LATENT // _OBL,),
        in_specs=[
            pl.BlockSpec((1, _N_GENES), lambda i: (0, 0)),
            pl.BlockSpec((_OBL, 1, 1), lambda i: (i, 0, 0)),
        ],
        out_specs=pl.BlockSpec((_OBL, 1, _N_GENES), lambda i: (i, 0, 0)),
        out_shape=jax.ShapeDtypeStruct((_N_LATENT, 1, _N_GENES),
                                       jnp.float32),
    )(w2, lat3)


def kernel(latent, genes_oi, W_height_slope, W_height_scale, W_height_shift,
           W_overall_slope):
    gather = _sc_gather()
    v3 = lambda w: w.reshape(_N_GENES // 8, 8, _N_DH)
    g_s = gather(v3(W_height_slope), genes_oi)
    g_c = gather(v3(W_height_scale), genes_oi)
    g_t = gather(v3(W_height_shift), genes_oi)
    lat2 = latent.reshape(_N_LATENT, 1)
    dh_t = _height(g_s.T, g_c.T, g_t.T, lat2)
    do_f = _overall(W_overall_slope.reshape(1, _N_GENES),
                    latent.reshape(_N_LATENT, 1, 1))
    dh = jnp.transpose(dh_t, (2, 1, 0))
    do = jnp.transpose(do_f, (2, 1, 0))
    return (dh, do)


# speedup vs baseline: 6.0171x; 1.0469x over previous
import functools

import jax
import jax.numpy as jnp
from jax import lax
from jax.experimental import pallas as pl
from jax.experimental.pallas import tpu as pltpu
from jax.experimental.pallas import tpu_sc as plsc

_N_GENES = 100000
_N_DH = 64
_N_LATENT = 50
_N_OI = 4096

_SC_CORES = 2
_SC_SUBCORES = 16
_NW = _SC_CORES * _SC_SUBCORES
_RPW = _N_OI // _NW


_L = 16


def _sc_gather_body(t_s, t_c, t_t, idx_hbm, out_s, out_c, out_t,
                    gidx, rows_s, rows_c, rows_t, sem_s, sem_c, sem_t):
    wid = lax.axis_index("s") * _SC_CORES + lax.axis_index("c")
    base = wid * _RPW
    pltpu.sync_copy(idx_hbm.at[pl.ds(base, _RPW)], gidx)
    lanes = lax.iota(jnp.int32, _L)

    def fire(i, carry):
        chunk = gidx[pl.ds((i // _L) * _L, _L)]
        g = jnp.sum(jnp.where(lanes == i % _L, chunk, 0))
        t = lax.shift_right_logical(g, 3)
        r = lax.bitwise_and(g, 7)
        pltpu.make_async_copy(
            t_s.at[t, pl.ds(r, 1)], rows_s.at[pl.ds(i, 1)], sem_s).start()
        pltpu.make_async_copy(
            t_c.at[t, pl.ds(r, 1)], rows_c.at[pl.ds(i, 1)], sem_c).start()
        pltpu.make_async_copy(
            t_t.at[t, pl.ds(r, 1)], rows_t.at[pl.ds(i, 1)], sem_t).start()
        return carry

    lax.fori_loop(0, _RPW, fire, 0)
    for rows, out, sem in ((rows_s, out_s, sem_s), (rows_c, out_c, sem_c),
                           (rows_t, out_t, sem_t)):
        pltpu.make_async_copy(out.at[pl.ds(0, _RPW)], rows, sem).wait()
        pltpu.sync_copy(rows, out.at[pl.ds(base, _RPW)])


@functools.cache
def _sc_gather():
  return pl.kernel(
    _sc_gather_body,
    out_type=[jax.ShapeDtypeStruct((_N_OI, _N_DH), jnp.float32)] * 3,
    mesh=plsc.VectorSubcoreMesh(
        core_axis_name="c", subcore_axis_name="s",
        num_cores=_SC_CORES, num_subcores=_SC_SUBCORES),
    scratch_types=[
        pltpu.VMEM((_RPW,), jnp.int32),
        pltpu.VMEM((_RPW, _N_DH), jnp.float32),
        pltpu.VMEM((_RPW, _N_DH), jnp.float32),
        pltpu.VMEM((_RPW, _N_DH), jnp.float32),
        pltpu.SemaphoreType.DMA,
        pltpu.SemaphoreType.DMA,
        pltpu.SemaphoreType.DMA,
    ],
    compiler_params=pltpu.CompilerParams(needs_layout_passes=False),
  )


def _height_body(s_ref, c_ref, t_ref, lat_ref, o_ref):
    e = jnp.exp(c_ref[...])[None, :, :]
    t = t_ref[...][None, :, :]
    s = s_ref[...][None, :, :]
    lat = lat_ref[...][:, :, None]
    x = e * lat + t
    o_ref[...] = s / (1.0 + jnp.exp(-x))


_HB = 512


def _height(gst, gct, gtt, lat2):
    return pl.pallas_call(
        _height_body,
        grid=(_N_OI // _HB,),
        in_specs=[
            pl.BlockSpec((_N_DH, _HB), lambda i: (0, i)),
            pl.BlockSpec((_N_DH, _HB), lambda i: (0, i)),
            pl.BlockSpec((_N_DH, _HB), lambda i: (0, i)),
            pl.BlockSpec((_N_LATENT, 1), lambda i: (0, 0)),
        ],
        out_specs=pl.BlockSpec((_N_LATENT, _N_DH, _HB),
                               lambda i: (0, 0, i)),
        out_shape=jax.ShapeDtypeStruct((_N_LATENT, _N_DH, _N_OI),
                                       jnp.float32),
    )(gst, gct, gtt, lat2)


def _overall_body(w_ref, lat_ref, o_ref):
    o_ref[...] = w_ref[...][None] * lat_ref[...]


_OBL = 10


def _overall(w2, lat3):
    return pl.pallas_call(
        _overall_body,
        grid=(_N_LATENT // _OBL,),
        in_specs=[
            pl.BlockSpec((1, _N_GENES), lambda i: (0, 0)),
            pl.BlockSpec((_OBL, 1, 1), lambda i: (i, 0, 0)),
        ],
        out_specs=pl.BlockSpec((_OBL, 1, _N_GENES), lambda i: (i, 0, 0)),
        out_shape=jax.ShapeDtypeStruct((_N_LATENT, 1, _N_GENES),
                                       jnp.float32),
    )(w2, lat3)


def kernel(latent, genes_oi, W_height_slope, W_height_scale, W_height_shift,
           W_overall_slope):
    v3 = lambda w: w.reshape(_N_GENES // 8, 8, _N_DH)
    g_s, g_c, g_t = _sc_gather()(
        v3(W_height_slope), v3(W_height_scale), v3(W_height_shift), genes_oi)
    lat2 = latent.reshape(_N_LATENT, 1)
    dh_t = _height(g_s.T, g_c.T, g_t.T, lat2)
    do_f = _overall(W_overall_slope.reshape(1, _N_GENES),
                    latent.reshape(_N_LATENT, 1, 1))
    dh = jnp.transpose(dh_t, (2, 1, 0))
    do = jnp.transpose(do_f, (2, 1, 0))
    return (dh, do)


# speedup vs baseline: 6.3780x; 1.0600x over previous
import functools

import jax
import jax.numpy as jnp
from jax import lax
from jax.experimental import pallas as pl
from jax.experimental.pallas import tpu as pltpu
from jax.experimental.pallas import tpu_sc as plsc

_N_GENES = 100000
_N_DH = 64
_N_LATENT = 50
_N_OI = 4096

_SC_CORES = 2
_SC_SUBCORES = 16
_NW = _SC_CORES * _SC_SUBCORES
_RPW = _N_OI // _NW


_L = 16


def _sc_gather_body(t_s, t_c, t_t, idx_hbm, out_s, out_c, out_t,
                    gidx, rows_s, rows_c, rows_t, sem_s, sem_c, sem_t):
    wid = lax.axis_index("s") * _SC_CORES + lax.axis_index("c")
    base = wid * _RPW
    pltpu.sync_copy(idx_hbm.at[pl.ds(base, _RPW)], gidx)
    lanes = lax.iota(jnp.int32, _L)

    def fire(i, carry):
        chunk = gidx[pl.ds((i // _L) * _L, _L)]
        g = jnp.sum(jnp.where(lanes == i % _L, chunk, 0))
        t = lax.shift_right_logical(g, 3)
        r = lax.bitwise_and(g, 7)
        pltpu.make_async_copy(
            t_s.at[t, pl.ds(r, 1)], rows_s.at[pl.ds(i, 1)], sem_s).start()
        pltpu.make_async_copy(
            t_c.at[t, pl.ds(r, 1)], rows_c.at[pl.ds(i, 1)], sem_c).start()
        pltpu.make_async_copy(
            t_t.at[t, pl.ds(r, 1)], rows_t.at[pl.ds(i, 1)], sem_t).start()
        return carry

    lax.fori_loop(0, _RPW, fire, 0)
    for rows, out, sem in ((rows_s, out_s, sem_s), (rows_c, out_c, sem_c),
                           (rows_t, out_t, sem_t)):
        pltpu.make_async_copy(out.at[pl.ds(0, _RPW)], rows, sem).wait()
        pltpu.sync_copy(rows, out.at[pl.ds(base, _RPW)])


@functools.cache
def _sc_gather():
  return pl.kernel(
    _sc_gather_body,
    out_type=[jax.ShapeDtypeStruct((_N_OI, _N_DH), jnp.float32)] * 3,
    mesh=plsc.VectorSubcoreMesh(
        core_axis_name="c", subcore_axis_name="s",
        num_cores=_SC_CORES, num_subcores=_SC_SUBCORES),
    scratch_types=[
        pltpu.VMEM((_RPW,), jnp.int32),
        pltpu.VMEM((_RPW, _N_DH), jnp.float32),
        pltpu.VMEM((_RPW, _N_DH), jnp.float32),
        pltpu.VMEM((_RPW, _N_DH), jnp.float32),
        pltpu.SemaphoreType.DMA,
        pltpu.SemaphoreType.DMA,
        pltpu.SemaphoreType.DMA,
    ],
    compiler_params=pltpu.CompilerParams(needs_layout_passes=False),
  )


def _height_body(s_ref, c_ref, t_ref, lat_ref, o_ref):
    e = jnp.exp(c_ref[...]).T[None, :, :]
    t = t_ref[...].T[None, :, :]
    s = s_ref[...].T[None, :, :]
    lat = lat_ref[...][:, :, None]
    x = e * lat + t
    o_ref[...] = s / (1.0 + jnp.exp(-x))


_HB = 512


def _height(gs, gc, gt, lat2):
    return pl.pallas_call(
        _height_body,
        grid=(_N_OI // _HB,),
        in_specs=[
            pl.BlockSpec((_HB, _N_DH), lambda i: (i, 0)),
            pl.BlockSpec((_HB, _N_DH), lambda i: (i, 0)),
            pl.BlockSpec((_HB, _N_DH), lambda i: (i, 0)),
            pl.BlockSpec((_N_LATENT, 1), lambda i: (0, 0)),
        ],
        out_specs=pl.BlockSpec((_N_LATENT, _N_DH, _HB),
                               lambda i: (0, 0, i)),
        out_shape=jax.ShapeDtypeStruct((_N_LATENT, _N_DH, _N_OI),
                                       jnp.float32),
    )(gs, gc, gt, lat2)


def _overall_body(w_ref, lat_ref, o_ref):
    o_ref[...] = w_ref[...][None] * lat_ref[...]


_OBL = 10


def _overall(w2, lat3):
    return pl.pallas_call(
        _overall_body,
        grid=(_N_LATENT // _OBL,),
        in_specs=[
            pl.BlockSpec((1, _N_GENES), lambda i: (0, 0)),
            pl.BlockSpec((_OBL, 1, 1), lambda i: (i, 0, 0)),
        ],
        out_specs=pl.BlockSpec((_OBL, 1, _N_GENES), lambda i: (i, 0, 0)),
        out_shape=jax.ShapeDtypeStruct((_N_LATENT, 1, _N_GENES),
                                       jnp.float32),
    )(w2, lat3)


def kernel(latent, genes_oi, W_height_slope, W_height_scale, W_height_shift,
           W_overall_slope):
    v3 = lambda w: w.reshape(_N_GENES // 8, 8, _N_DH)
    g_s, g_c, g_t = _sc_gather()(
        v3(W_height_slope), v3(W_height_scale), v3(W_height_shift), genes_oi)
    lat2 = latent.reshape(_N_LATENT, 1)
    dh_t = _height(g_s, g_c, g_t, lat2)
    do_f = _overall(W_overall_slope.reshape(1, _N_GENES),
                    latent.reshape(_N_LATENT, 1, 1))
    dh = jnp.transpose(dh_t, (2, 1, 0))
    do = jnp.transpose(do_f, (2, 1, 0))
    return (dh, do)
